# Initial kernel scaffold; baseline (speedup 1.0000x reference)
import jax
import jax.numpy as jnp
from jax.experimental import pallas as pl


def kernel(states, actions, returns_to_go, time_steps, padding_mask, timestep_table, state_W, state_b, return_W, return_b, act_W, act_b, action_pos_table):
    raise NotImplementedError("write your pallas kernel here")


# speedup vs baseline: 2.7762x; 2.7762x over previous
import functools

import jax
import jax.numpy as jnp
from jax import lax
from jax.experimental import pallas as pl
from jax.experimental.pallas import tpu as pltpu
from jax.experimental.pallas import tpu_sc as plsc

_LB = 128
_CH = 32


def _sc_gather_rows(table, idx):
    n = idx.shape[0]
    d = table.shape[1]
    info = plsc.get_sparse_core_info()
    nw = info.num_cores * info.num_subcores
    rows_w = n // nw
    nch = rows_w // _CH
    mesh = plsc.VectorSubcoreMesh(core_axis_name="c", subcore_axis_name="s")

    @functools.partial(
        pl.kernel,
        mesh=mesh,
        out_type=jax.ShapeDtypeStruct((n, d), jnp.float32),
        scratch_types=[
            pltpu.VMEM((_CH,), jnp.int32),
            pltpu.VMEM((_CH, d), jnp.float32),
            pltpu.SemaphoreType.DMA,
        ],
    )
    def gather_k(table_hbm, idx_hbm, out_hbm, idx_v, rows_v, sem):
        wid = lax.axis_index("s") * info.num_cores + lax.axis_index("c")
        base = wid * rows_w

        def body(i, carry):
            off = base + i * _CH
            pltpu.sync_copy(idx_hbm.at[pl.ds(off, _CH)], idx_v)
            pltpu.async_copy(table_hbm.at[idx_v], rows_v, sem).wait()
            pltpu.sync_copy(rows_v, out_hbm.at[pl.ds(off, _CH)])
            return carry

        lax.fori_loop(0, nch, body, 0)

    return gather_k(table, idx)


def _tc_assemble(te3, states, aug, state_w, params):
    b, l, d = te3.shape
    sdim = states.shape[-1]
    a = aug.shape[-1] - 1
    spread = 2 + a

    def body(te_ref, st_ref, ag_ref, sw_ref, par_ref, out_ref):
        te = te_ref[0]
        te2 = te + te
        p = par_ref[...]
        s_emb = (
            jnp.dot(st_ref[0], sw_ref[...], preferred_element_type=jnp.float32)
            + p[2][None, :] + te2
        )
        ag = ag_ref[0]
        r_emb = ag[:, 0][:, None] * p[0][None, :] + p[1][None, :] + te2
        planes = [r_emb, s_emb]
        for j in range(a):
            planes.append(
                ag[:, 1 + j][:, None] * p[3][None, :] + p[4][None, :]
                + te + p[5 + j][None, :]
            )
        out_ref[0] = jnp.stack(planes, axis=1)

    return pl.pallas_call(
        body,
        grid=(b, l // _LB),
        in_specs=[
            pl.BlockSpec((1, _LB, d), lambda i, j: (i, j, 0)),
            pl.BlockSpec((1, _LB, sdim), lambda i, j: (i, j, 0)),
            pl.BlockSpec((1, _LB, 1 + a), lambda i, j: (i, j, 0)),
            pl.BlockSpec((sdim, d), lambda i, j: (0, 0)),
            pl.BlockSpec((5 + a, d), lambda i, j: (0, 0)),
        ],
        out_specs=pl.BlockSpec((1, _LB, spread, d), lambda i, j: (i, j, 0, 0)),
        out_shape=jax.ShapeDtypeStruct((b, l, spread, d), jnp.float32),
        compiler_params=pltpu.CompilerParams(
            dimension_semantics=("parallel", "parallel"),
        ),
    )(te3, states, aug, state_w, params)


def kernel(states, actions, returns_to_go, time_steps, padding_mask,
           timestep_table, state_W, state_b, return_W, return_b,
           act_W, act_b, action_pos_table):
    b, l, sdim = states.shape
    a = actions.shape[-1]
    d = timestep_table.shape[1]
    spread = 2 + a

    te = _sc_gather_rows(
        timestep_table, time_steps.reshape(b * l).astype(jnp.int32))
    te3 = te.reshape(b, l, d)

    aug = jnp.concatenate([returns_to_go[..., None], actions], axis=-1)
    params = jnp.concatenate(
        [return_W[None], return_b[None], state_b[None],
         act_W[None], act_b[None], action_pos_table], axis=0)

    out = _tc_assemble(te3, states, aug, state_W, params)
    embeds = out.reshape(b, l * spread, d)
    pm = jnp.repeat(padding_mask, spread, axis=1)
    return embeds, pm
